# baseline (speedup 1.0000x reference)
import functools

import jax
import jax.numpy as jnp
from jax import lax
from jax.experimental import pallas as pl
from jax.experimental.pallas import tpu as pltpu
from jax.experimental.pallas import tpu_sc as plsc

N = 320000
C = 128
R = 10000
NC = 2
NS = 16
NW = NC * NS
RPW = N // NW
BLK = 40
NBLK = RPW // BLK
NBUF = 5
LOOKAHEAD = 3
NITER = NBLK // NBUF
NSTAGE = 3
GRP = NBUF * BLK
NGRP = N // GRP
ACC_R = 10240
OUT_SLICE = ACC_R // NS
LANES = 16


def _sc_segment_sum(sem, w3, idx3):
    mesh = plsc.VectorSubcoreMesh(core_axis_name="c", subcore_axis_name="s")

    @functools.partial(
        pl.kernel,
        mesh=mesh,
        out_type=jax.ShapeDtypeStruct((NC, ACC_R, C), jnp.float32),
        scratch_types=[
            pltpu.VMEM((NBUF, BLK, C), jnp.float32),
            pltpu.VMEM((NSTAGE, NBUF, BLK), jnp.float32),
            pltpu.VMEM((NSTAGE, NBUF, BLK), jnp.int32),
            pltpu.VMEM_SHARED((ACC_R, C), jnp.float32),
        ] + [pltpu.SemaphoreType.DMA] * (2 * NBUF)
          + [pltpu.SemaphoreType.DMA((NSTAGE,))],
    )
    def k(sem_hbm, w_hbm, idx_hbm, out_hbm, sem_buf, w_stage, idx_stage,
          acc, *sems):
        lsem = sems[:NBUF]
        ssem = sems[NBUF:2 * NBUF]
        gsem_arr = sems[2 * NBUF]
        c = lax.axis_index("c")
        s = lax.axis_index("s")
        wid = s * NC + c
        row0 = wid * RPW
        grp0 = wid * NITER

        def issue_sem_load(jj, b):
            pltpu.async_copy(sem_hbm.at[pl.ds(row0 + jj * BLK, BLK)],
                             sem_buf.at[b], lsem[b])

        def wait_sem_load(jj, b):
            pltpu.make_async_copy(sem_hbm.at[pl.ds(row0 + jj * BLK, BLK)],
                                  sem_buf.at[b], lsem[b]).wait()

        def issue_stage(gi, d):
            pltpu.async_copy(w_hbm.at[grp0 + gi], w_stage.at[d],
                             gsem_arr.at[d])
            pltpu.async_copy(idx_hbm.at[grp0 + gi], idx_stage.at[d],
                             gsem_arr.at[d])

        def wait_stage(gi, d):
            pltpu.make_async_copy(w_hbm.at[grp0 + gi], w_stage.at[d],
                                  gsem_arr.at[d]).wait()
            pltpu.make_async_copy(idx_hbm.at[grp0 + gi], idx_stage.at[d],
                                  gsem_arr.at[d]).wait()

        def start_scatter(d, u):
            pltpu.async_copy(sem_buf.at[u], acc.at[idx_stage.at[d, u]],
                             ssem[u], add=True)

        def wait_scatter(d, u):
            pltpu.make_async_copy(sem_buf.at[u], acc.at[idx_stage.at[d, u]],
                                  ssem[u]).wait()

        def scale_block(d, u):
            for off, l0 in ((0, 0), (16, 0), (24, 8)):
                wv = w_stage[d, u, pl.ds(off, LANES)]
                for l in range(l0, LANES):
                    r = off + l
                    wb = jnp.broadcast_to(wv[l], (LANES,))
                    for h in range(C // LANES):
                        sl = pl.ds(h * LANES, LANES)
                        sem_buf[u, r, sl] = sem_buf[u, r, sl] * wb

        zero16 = jnp.zeros((LANES,), jnp.float32)
        for i in range(BLK):
            for h in range(C // LANES):
                sem_buf[0, i, pl.ds(h * LANES, LANES)] = zero16
        for t in range(OUT_SLICE // BLK):
            pltpu.async_copy(sem_buf.at[0],
                             acc.at[pl.ds(s * OUT_SLICE + t * BLK, BLK)],
                             ssem[0])
        for t in range(OUT_SLICE // BLK):
            pltpu.make_async_copy(sem_buf.at[0],
                                  acc.at[pl.ds(s * OUT_SLICE + t * BLK, BLK)],
                                  ssem[0]).wait()
        plsc.subcore_barrier()

        issue_stage(0, 0)
        issue_stage(1, 1)
        for b in range(LOOKAHEAD):
            issue_sem_load(b, b)

        def body(i, carry):
            d = lax.rem(i, NSTAGE)
            wait_stage(i, d)
            for u in range(NBUF):
                j = NBUF * i + u
                wait_sem_load(j, u)
                scale_block(d, u)
                start_scatter(d, u)
                tb = (u + LOOKAHEAD) % NBUF
                if u + LOOKAHEAD < NBUF:
                    @pl.when(i >= 1)
                    def _():
                        wait_scatter(d, tb)

                    issue_sem_load(j + LOOKAHEAD, tb)
                else:
                    @pl.when(i <= NITER - 2)
                    def _():
                        wait_scatter(d, tb)
                        issue_sem_load(j + LOOKAHEAD, tb)

            @pl.when(i <= NITER - 3)
            def _():
                issue_stage(i + 2, lax.rem(i + 2, NSTAGE))

            return carry

        lax.fori_loop(0, NITER, body, 0)

        for b in range(NBUF):
            wait_scatter(0, b)
        plsc.subcore_barrier()
        pltpu.sync_copy(
            acc.at[pl.ds(s * OUT_SLICE, OUT_SLICE)],
            out_hbm.at[c].at[pl.ds(s * OUT_SLICE, OUT_SLICE)],
        )

    return k(sem, w3, idx3)


def _tc_combine(partial):
    def body(a_ref, b_ref, o_ref):
        o_ref[...] = a_ref[...] + b_ref[...]

    blk = 1000
    return pl.pallas_call(
        body,
        grid=(R // blk,),
        in_specs=[
            pl.BlockSpec((blk, C), lambda i: (i, 0)),
            pl.BlockSpec((blk, C), lambda i: (i, 0)),
        ],
        out_specs=pl.BlockSpec((blk, C), lambda i: (i, 0)),
        out_shape=jax.ShapeDtypeStruct((R, C), jnp.float32),
    )(partial[0], partial[1])


def kernel(semantics, weights, ray_indices, num_rays):
    idx = jnp.minimum(ray_indices,
                      jnp.asarray(num_rays, ray_indices.dtype) - 1)
    w3 = weights.reshape(NGRP, NBUF, BLK)
    idx3 = idx.reshape(NGRP, NBUF, BLK)
    partial = _sc_segment_sum(semantics, w3, idx3)
    return _tc_combine(partial[:, :R])


# speedup vs baseline: 1.0736x; 1.0736x over previous
import functools

import jax
import jax.numpy as jnp
from jax import lax
from jax.experimental import pallas as pl
from jax.experimental.pallas import tpu as pltpu
from jax.experimental.pallas import tpu_sc as plsc

N = 320000
C = 128
R = 10000
NC = 2
NS = 16
NW = NC * NS
RPW = N // NW
BLK = 40
NBLK = RPW // BLK
NBUF = 5
LOOKAHEAD = 3
NITER = NBLK // NBUF
ACC_R = 10240
OUT_SLICE = ACC_R // NS
LANES = 16
WPAD = 48


def _sc_segment_sum(sem, wflat, idxflat):
    mesh = plsc.VectorSubcoreMesh(core_axis_name="c", subcore_axis_name="s")

    @functools.partial(
        pl.kernel,
        mesh=mesh,
        out_type=jax.ShapeDtypeStruct((NC, ACC_R, C), jnp.float32),
        scratch_types=[
            pltpu.VMEM((NBUF, BLK, C), jnp.float32),
            pltpu.VMEM((NBUF, WPAD), jnp.float32),
            pltpu.VMEM((NBUF, BLK), jnp.int32),
            pltpu.VMEM_SHARED((ACC_R, C), jnp.float32),
        ] + [pltpu.SemaphoreType.DMA] * (2 * NBUF),
    )
    def k(sem_hbm, w_hbm, idx_hbm, out_hbm, sem_buf, w_buf, idx_buf, acc,
          *sems):
        lsem = sems[:NBUF]
        ssem = sems[NBUF:]
        c = lax.axis_index("c")
        s = lax.axis_index("s")
        wid = s * NC + c
        row0 = wid * RPW

        def issue_loads(jj, b):
            base = row0 + jj * BLK
            pltpu.async_copy(sem_hbm.at[pl.ds(base, BLK)], sem_buf.at[b],
                             lsem[b])
            pltpu.async_copy(idx_hbm.at[pl.ds(base, BLK)], idx_buf.at[b],
                             lsem[b])
            pltpu.async_copy(w_hbm.at[pl.ds(base, BLK)],
                             w_buf.at[b, pl.ds(0, BLK)], lsem[b])

        def wait_loads(jj, b):
            base = row0 + jj * BLK
            pltpu.make_async_copy(sem_hbm.at[pl.ds(base, BLK)],
                                  sem_buf.at[b], lsem[b]).wait()
            pltpu.make_async_copy(idx_hbm.at[pl.ds(base, BLK)],
                                  idx_buf.at[b], lsem[b]).wait()
            pltpu.make_async_copy(w_hbm.at[pl.ds(base, BLK)],
                                  w_buf.at[b, pl.ds(0, BLK)], lsem[b]).wait()

        def start_scatter(b):
            pltpu.async_copy(sem_buf.at[b], acc.at[idx_buf.at[b]], ssem[b],
                             add=True)

        def wait_scatter(b):
            pltpu.make_async_copy(sem_buf.at[b], acc.at[idx_buf.at[b]],
                                  ssem[b]).wait()

        def scale_block(b):
            for g in range((BLK + LANES - 1) // LANES):
                wv = w_buf[b, pl.ds(g * LANES, LANES)]
                for l in range(min(LANES, BLK - g * LANES)):
                    r = g * LANES + l
                    wb = jnp.broadcast_to(wv[l], (LANES,))
                    for h in range(C // LANES):
                        sl = pl.ds(h * LANES, LANES)
                        sem_buf[b, r, sl] = sem_buf[b, r, sl] * wb

        zero16 = jnp.zeros((LANES,), jnp.float32)
        for i in range(BLK):
            for h in range(C // LANES):
                sem_buf[0, i, pl.ds(h * LANES, LANES)] = zero16
        for t in range(OUT_SLICE // BLK):
            pltpu.sync_copy(sem_buf.at[0],
                            acc.at[pl.ds(s * OUT_SLICE + t * BLK, BLK)])
        plsc.subcore_barrier()

        for b in range(LOOKAHEAD):
            issue_loads(b, b)

        def body(i, carry):
            for u in range(NBUF):
                j = NBUF * i + u
                wait_loads(j, u)
                tb = (u + LOOKAHEAD) % NBUF
                if u + LOOKAHEAD < NBUF:
                    @pl.when(i >= 1)
                    def _():
                        wait_scatter(tb)

                    issue_loads(j + LOOKAHEAD, tb)
                else:
                    @pl.when(i <= NITER - 2)
                    def _():
                        wait_scatter(tb)
                        issue_loads(j + LOOKAHEAD, tb)
                scale_block(u)
                start_scatter(u)
            return carry

        lax.fori_loop(0, NITER, body, 0)

        for b in range(NBUF):
            wait_scatter(b)
        plsc.subcore_barrier()
        pltpu.sync_copy(
            acc.at[pl.ds(s * OUT_SLICE, OUT_SLICE)],
            out_hbm.at[c].at[pl.ds(s * OUT_SLICE, OUT_SLICE)],
        )

    return k(sem, wflat, idxflat)


def _tc_combine(partial):
    def body(a_ref, b_ref, o_ref):
        o_ref[...] = a_ref[...] + b_ref[...]

    blk = 1000
    return pl.pallas_call(
        body,
        grid=(R // blk,),
        in_specs=[
            pl.BlockSpec((blk, C), lambda i: (i, 0)),
            pl.BlockSpec((blk, C), lambda i: (i, 0)),
        ],
        out_specs=pl.BlockSpec((blk, C), lambda i: (i, 0)),
        out_shape=jax.ShapeDtypeStruct((R, C), jnp.float32),
    )(partial[0], partial[1])


def kernel(semantics, weights, ray_indices, num_rays):
    idx = jnp.minimum(ray_indices,
                      jnp.asarray(num_rays, ray_indices.dtype) - 1)
    partial = _sc_segment_sum(semantics, weights.reshape(N), idx)
    return _tc_combine(partial[:, :R])
